# baseline (speedup 1.0000x reference)
import jax
import jax.numpy as jnp
from jax.experimental import pallas as pl

N_ = 512
D_ = 512
T_ = 512
K_ = 6
BT_ = 128


def _codebook_kernel(z_ref, c_ref, oh_ref, idx_ref):
    z = z_ref[...]
    c = c_ref[...]
    zt = z.T
    ct = c.T

    scores = jax.lax.dot_general(
        c, zt, (((1,), (0,)), ((), ())), preferred_element_type=jnp.float32
    )
    cn = jnp.sum(c * c, axis=1, keepdims=True)
    dist = cn - 2.0 * scores

    riota = jax.lax.broadcasted_iota(jnp.int32, (N_, BT_), 0)

    key = jax.lax.bitcast_convert_type(dist + 64.0, jnp.int32)
    key = (key & jnp.int32(~511)) | riota
    cand = []
    for _ in range(K_):
        m = jnp.min(key, axis=0, keepdims=True)
        cand.append(m & 511)
        key = jnp.where(key == m, jnp.int32(0x7FFFFFFF), key)

    hi = ct.astype(jnp.bfloat16)
    r1 = ct - hi.astype(jnp.float32)
    mid = r1.astype(jnp.bfloat16)
    lo = (r1 - mid.astype(jnp.float32)).astype(jnp.bfloat16)
    cstack = jnp.concatenate([hi, mid, lo], axis=1)

    bestd = None
    besti = None
    for r in range(K_):
        ohT = (riota == cand[r]).astype(jnp.bfloat16)
        oh3 = jnp.concatenate([ohT, ohT, ohT], axis=0)
        cvt = jax.lax.dot_general(
            cstack, oh3, (((1,), (0,)), ((), ())),
            preferred_element_type=jnp.float32)
        diff = cvt - zt
        dsq = diff * diff
        sc = []
        for ch in range(4):
            base = ch * 128
            p = dsq[base:base + 8, :]
            for v in range(1, 16):
                p = p + dsq[base + v * 8: base + (v + 1) * 8, :]
            t1 = (p[0:1, :] + p[4:5, :]) + (p[2:3, :] + p[6:7, :])
            t2 = (p[1:2, :] + p[5:6, :]) + (p[3:4, :] + p[7:8, :])
            sc.append(t1 + t2)
        s = ((sc[0] + sc[1]) + sc[2]) + sc[3]
        dr = jnp.sqrt(s)
        if r == 0:
            bestd, besti = dr, cand[r]
        else:
            take = (dr < bestd) | ((dr == bestd) & (cand[r] < besti))
            bestd = jnp.where(take, dr, bestd)
            besti = jnp.where(take, cand[r], besti)

    idx_ref[...] = besti
    ciota = jax.lax.broadcasted_iota(jnp.int32, (BT_, N_), 1)
    oh_ref[...] = (ciota == besti.T).astype(jnp.float32)


def kernel(batch_z, codebook_vectors):
    z = batch_z.reshape(-1, D_)
    one_hot, idx = pl.pallas_call(
        _codebook_kernel,
        grid=(T_ // BT_,),
        in_specs=[
            pl.BlockSpec((BT_, D_), lambda i: (i, 0)),
            pl.BlockSpec((N_, D_), lambda i: (0, 0)),
        ],
        out_specs=(
            pl.BlockSpec((BT_, N_), lambda i: (i, 0)),
            pl.BlockSpec((1, BT_), lambda i: (0, i)),
        ),
        out_shape=(
            jax.ShapeDtypeStruct((T_, N_), jnp.float32),
            jax.ShapeDtypeStruct((1, T_), jnp.int32),
        ),
    )(z, codebook_vectors)
    return one_hot, idx.reshape(-1)


# speedup vs baseline: 1.7095x; 1.7095x over previous
import jax
import jax.numpy as jnp
from jax.experimental import pallas as pl

N_ = 512
D_ = 512
T_ = 512
K_ = 6
BT_ = 512


def _codebook_kernel(z_ref, c_ref, oh_ref, idx_ref):
    z = z_ref[...]
    c = c_ref[...]
    zt = z.T
    ct = c.T

    scores = jax.lax.dot_general(
        c, zt, (((1,), (0,)), ((), ())), preferred_element_type=jnp.float32
    )
    cn = jnp.sum(c * c, axis=1, keepdims=True)
    dist = cn - 2.0 * scores

    riota = jax.lax.broadcasted_iota(jnp.int32, (N_, BT_), 0)

    key = jax.lax.bitcast_convert_type(dist + 64.0, jnp.int32)
    key = (key & jnp.int32(~511)) | riota
    cand = []
    for _ in range(K_):
        m = jnp.min(key, axis=0, keepdims=True)
        cand.append(m & 511)
        key = jnp.where(key == m, jnp.int32(0x7FFFFFFF), key)

    hi = ct.astype(jnp.bfloat16)
    r1 = ct - hi.astype(jnp.float32)
    mid = r1.astype(jnp.bfloat16)
    lo = (r1 - mid.astype(jnp.float32)).astype(jnp.bfloat16)
    cstack = jnp.concatenate([hi, mid, lo], axis=1)

    bestd = None
    besti = None
    for r in range(K_):
        ohT = (riota == cand[r]).astype(jnp.bfloat16)
        oh3 = jnp.concatenate([ohT, ohT, ohT], axis=0)
        cvt = jax.lax.dot_general(
            cstack, oh3, (((1,), (0,)), ((), ())),
            preferred_element_type=jnp.float32)
        diff = cvt - zt
        dsq = diff * diff
        sc = []
        for ch in range(4):
            base = ch * 128
            p = dsq[base:base + 8, :]
            for v in range(1, 16):
                p = p + dsq[base + v * 8: base + (v + 1) * 8, :]
            t1 = (p[0:1, :] + p[4:5, :]) + (p[2:3, :] + p[6:7, :])
            t2 = (p[1:2, :] + p[5:6, :]) + (p[3:4, :] + p[7:8, :])
            sc.append(t1 + t2)
        s = ((sc[0] + sc[1]) + sc[2]) + sc[3]
        dr = jnp.sqrt(s)
        if r == 0:
            bestd, besti = dr, cand[r]
        else:
            take = (dr < bestd) | ((dr == bestd) & (cand[r] < besti))
            bestd = jnp.where(take, dr, bestd)
            besti = jnp.where(take, cand[r], besti)

    idx_ref[...] = besti
    ciota = jax.lax.broadcasted_iota(jnp.int32, (BT_, N_), 1)
    oh_ref[...] = (ciota == besti.T).astype(jnp.float32)


def kernel(batch_z, codebook_vectors):
    z = batch_z.reshape(-1, D_)
    one_hot, idx = pl.pallas_call(
        _codebook_kernel,
        out_shape=(
            jax.ShapeDtypeStruct((T_, N_), jnp.float32),
            jax.ShapeDtypeStruct((1, T_), jnp.int32),
        ),
    )(z, codebook_vectors)
    return one_hot, idx.reshape(-1)


# speedup vs baseline: 1.7313x; 1.0127x over previous
import jax
import jax.numpy as jnp
from jax.experimental import pallas as pl

N_ = 512
D_ = 512
T_ = 512
K_ = 6
BT_ = 512


def _codebook_kernel(z_ref, c_ref, oh_ref, idx_ref):
    z = z_ref[...]
    c = c_ref[...]
    zt = z.T
    ct = c.T

    scores = jax.lax.dot_general(
        c, zt, (((1,), (0,)), ((), ())), preferred_element_type=jnp.float32
    )
    cn = jnp.sum(c * c, axis=1, keepdims=True)
    dist = cn - 2.0 * scores

    riota = jax.lax.broadcasted_iota(jnp.int32, (N_, BT_), 0)

    key = jax.lax.bitcast_convert_type(dist + 64.0, jnp.int32)
    key = (key & jnp.int32(~511)) | riota
    cand = []
    for _ in range(K_):
        m = jnp.min(key, axis=0, keepdims=True)
        cand.append(m & 511)
        key = jnp.where(key == m, jnp.int32(0x7FFFFFFF), key)

    hi = ct.astype(jnp.bfloat16)
    r1 = ct - hi.astype(jnp.float32)
    mid = r1.astype(jnp.bfloat16)
    lo = (r1 - mid.astype(jnp.float32)).astype(jnp.bfloat16)
    cstack = jnp.concatenate([hi, mid, lo], axis=1)

    riota3 = jax.lax.broadcasted_iota(jnp.int32, (3 * N_, BT_), 0) & 511

    bestd = None
    besti = None
    for r in range(K_):
        oh3 = (riota3 == cand[r]).astype(jnp.bfloat16)
        cvt = jax.lax.dot_general(
            cstack, oh3, (((1,), (0,)), ((), ())),
            preferred_element_type=jnp.float32)
        diff = cvt - zt
        dsq = diff * diff
        sc = []
        for ch in range(4):
            base = ch * 128
            p = dsq[base:base + 8, :]
            for v in range(1, 16):
                p = p + dsq[base + v * 8: base + (v + 1) * 8, :]
            t1 = (p[0:1, :] + p[4:5, :]) + (p[2:3, :] + p[6:7, :])
            t2 = (p[1:2, :] + p[5:6, :]) + (p[3:4, :] + p[7:8, :])
            sc.append(t1 + t2)
        s = ((sc[0] + sc[1]) + sc[2]) + sc[3]
        dr = jnp.sqrt(s)
        if r == 0:
            bestd, besti = dr, cand[r]
        else:
            take = (dr < bestd) | ((dr == bestd) & (cand[r] < besti))
            bestd = jnp.where(take, dr, bestd)
            besti = jnp.where(take, cand[r], besti)

    idx_ref[...] = besti
    ciota = jax.lax.broadcasted_iota(jnp.int32, (BT_, N_), 1)
    oh_ref[...] = (ciota == besti.T).astype(jnp.float32)


def kernel(batch_z, codebook_vectors):
    z = batch_z.reshape(-1, D_)
    one_hot, idx = pl.pallas_call(
        _codebook_kernel,
        out_shape=(
            jax.ShapeDtypeStruct((T_, N_), jnp.float32),
            jax.ShapeDtypeStruct((1, T_), jnp.int32),
        ),
    )(z, codebook_vectors)
    return one_hot, idx.reshape(-1)
